# baseline (speedup 1.0000x reference)
import jax
import jax.numpy as jnp
from jax import lax
from jax.experimental import pallas as pl
from jax.experimental.pallas import tpu as pltpu
from jax.experimental.pallas import tpu_sc as plsc

N = 10000
E = 320000
D = 128
OUT = 1024

NC = 2
NS = 16
NT = NC * NS
EPT = E // NT
KW = 125
NWIN = EPT // KW
PKW = 50
GW = 20
NCH = EPT // (GW * PKW)
NB = 4
LA = 2
NRT = 10240
RPT = NRT // NS
NPAD = 10240
DSL = NPAD // NS

_mesh = plsc.VectorSubcoreMesh(
    core_axis_name="c", subcore_axis_name="s", num_cores=NC, num_subcores=NS
)


def _deg_body(dsts_hbm, zflat_hbm, out_hbm, dstv, onesv, dsem, deg_sh):
    c = lax.axis_index("c")
    s = lax.axis_index("s")
    chunk = c * NS + s
    pltpu.sync_copy(dsts_hbm.at[chunk], dstv)
    for i in range(128 // 16):
        onesv[pl.ds(i * 16, 16)] = jnp.ones((16,), jnp.float32)
    pltpu.sync_copy(zflat_hbm, deg_sh.at[pl.ds(s * DSL, DSL)])
    plsc.subcore_barrier()

    ones_kw = onesv.at[pl.ds(0, KW)]

    def fire(w, tok):
        pltpu.async_copy(ones_kw, deg_sh.at[dstv.at[w]], dsem, add=True)
        return tok

    def drain(w, tok):
        pltpu.make_async_copy(ones_kw, deg_sh.at[dstv.at[w]], dsem).wait()
        return tok

    lax.fori_loop(0, NWIN, fire, 0)
    lax.fori_loop(0, NWIN, drain, 0)
    plsc.subcore_barrier()
    pltpu.sync_copy(
        deg_sh.at[pl.ds(s * DSL, DSL)], out_hbm.at[c].at[pl.ds(s * DSL, DSL)]
    )


_deg_call = pl.kernel(
    _deg_body,
    out_type=jax.ShapeDtypeStruct((NC, NPAD), jnp.float32),
    mesh=_mesh,
    scratch_types=[
        pltpu.VMEM((NWIN, KW), jnp.int32),
        pltpu.VMEM((128,), jnp.float32),
        pltpu.SemaphoreType.DMA,
        pltpu.VMEM_SHARED((NPAD,), jnp.float32),
    ],
)


def _prop_body(g_hbm, srcs_hbm, dsts_hbm, zrows_hbm, out_hbm, sbuf, dbuf, buf, gsem, ssem, acc):
    c = lax.axis_index("c")
    s = lax.axis_index("s")
    tchunk = c * NS + s
    base = s * RPT
    pltpu.sync_copy(zrows_hbm, acc.at[pl.ds(base, RPT)])
    plsc.subcore_barrier()

    def gissue(wl, b):
        pltpu.async_copy(g_hbm.at[sbuf.at[wl]], buf.at[b], gsem.at[b])

    def gwait(wl, b):
        pltpu.make_async_copy(g_hbm.at[sbuf.at[wl]], buf.at[b], gsem.at[b]).wait()

    def sissue(wl, b):
        pltpu.async_copy(buf.at[b], acc.at[dbuf.at[wl]], ssem.at[b], add=True)

    def swait(wl, b):
        pltpu.make_async_copy(buf.at[b], acc.at[dbuf.at[wl]], ssem.at[b]).wait()

    def chunk_body(j, tok):
        pltpu.sync_copy(srcs_hbm.at[tchunk].at[j], sbuf)
        pltpu.sync_copy(dsts_hbm.at[tchunk].at[j], dbuf)
        for p in range(LA):
            gissue(p, p % NB)
        for wl in range(GW):
            b = wl % NB
            gwait(wl, b)
            sissue(wl, b)
            v = wl + LA
            if v < GW:
                bv = v % NB
                if v >= NB:
                    swait(v - NB, bv)
                gissue(v, bv)
        for wl in range(GW - NB, GW):
            if wl >= 0:
                swait(wl, wl % NB)
        return tok

    lax.fori_loop(0, NCH, chunk_body, 0)
    plsc.subcore_barrier()
    pltpu.sync_copy(acc.at[pl.ds(base, RPT)], out_hbm.at[c].at[pl.ds(base, RPT)])


_prop_call = pl.kernel(
    _prop_body,
    out_type=jax.ShapeDtypeStruct((NC, NRT, D), jnp.float32),
    mesh=_mesh,
    scratch_types=[
        pltpu.VMEM((GW, PKW), jnp.int32),
        pltpu.VMEM((GW, PKW), jnp.int32),
        pltpu.VMEM((NB, PKW, D), jnp.float32),
        pltpu.SemaphoreType.DMA((NB,)),
        pltpu.SemaphoreType.DMA((NB,)),
        pltpu.VMEM_SHARED((NRT, D), jnp.float32),
    ],
)


def _prep_kernel(degp_ref, x_ref, g_ref, dinv_ref, dinv2_ref):
    degb = degp_ref[0] + degp_ref[1] + 1.0
    dinv = lax.rsqrt(degb)
    dinv_ref[...] = dinv
    dinv2_ref[...] = 1.0 / degb
    g_ref[...] = dinv * x_ref[...]


def _prep(degp, x):
    RB = 400
    grid = N // RB
    return pl.pallas_call(
        _prep_kernel,
        grid=(grid,),
        in_specs=[
            pl.BlockSpec((NC, RB, 1), lambda r: (0, r, 0)),
            pl.BlockSpec((RB, D), lambda r: (r, 0)),
        ],
        out_specs=[
            pl.BlockSpec((RB, D), lambda r: (r, 0)),
            pl.BlockSpec((RB, 1), lambda r: (r, 0)),
            pl.BlockSpec((RB, 1), lambda r: (r, 0)),
        ],
        out_shape=[
            jax.ShapeDtypeStruct((N, D), jnp.float32),
            jax.ShapeDtypeStruct((N, 1), jnp.float32),
            jax.ShapeDtypeStruct((N, 1), jnp.float32),
        ],
    )(degp, x)


def _mid_kernel(tp_ref, g_ref, dinv2_ref, o_ref):
    o_ref[...] = dinv2_ref[...] * (tp_ref[0] + tp_ref[1] + g_ref[...])


def _mid(tp, g, dinv2):
    RB = 400
    grid = N // RB
    return pl.pallas_call(
        _mid_kernel,
        grid=(grid,),
        in_specs=[
            pl.BlockSpec((NC, RB, D), lambda r: (0, r, 0)),
            pl.BlockSpec((RB, D), lambda r: (r, 0)),
            pl.BlockSpec((RB, 1), lambda r: (r, 0)),
        ],
        out_specs=pl.BlockSpec((RB, D), lambda r: (r, 0)),
        out_shape=jax.ShapeDtypeStruct((N, D), jnp.float32),
    )(tp, g, dinv2)


def _round_kernel(tp_ref, g_ref, dinv_ref, w_ref, b_ref, o_ref):
    dinv = dinv_ref[...]
    h = dinv * (tp_ref[0] + tp_ref[1] + g_ref[...])
    hw = jnp.dot(h, w_ref[...], preferred_element_type=jnp.float32) + b_ref[...]
    o_ref[...] = dinv * hw


def _round(tp, g, dinv, w, b):
    RB = 1000
    grid = N // RB
    return pl.pallas_call(
        _round_kernel,
        grid=(grid,),
        in_specs=[
            pl.BlockSpec((NC, RB, D), lambda r: (0, r, 0)),
            pl.BlockSpec((RB, D), lambda r: (r, 0)),
            pl.BlockSpec((RB, 1), lambda r: (r, 0)),
            pl.BlockSpec((D, D), lambda r: (0, 0)),
            pl.BlockSpec((1, D), lambda r: (0, 0)),
        ],
        out_specs=pl.BlockSpec((RB, D), lambda r: (r, 0)),
        out_shape=jax.ShapeDtypeStruct((N, D), jnp.float32),
    )(tp, g, dinv, w, b)


def _final_kernel(tp_ref, g_ref, dinv_ref, w3_ref, b3_ref, w4_ref, b4_ref, o_ref):
    h = dinv_ref[...] * (tp_ref[0] + tp_ref[1] + g_ref[...])
    h3 = jnp.dot(h, w3_ref[...], preferred_element_type=jnp.float32) + b3_ref[...]
    o_ref[...] = (
        jnp.dot(h3, w4_ref[...], preferred_element_type=jnp.float32) + b4_ref[...]
    )


def _final(tp, g, dinv, w3, b3, w4, b4):
    RB = 400
    grid = N // RB
    return pl.pallas_call(
        _final_kernel,
        grid=(grid,),
        in_specs=[
            pl.BlockSpec((NC, RB, D), lambda r: (0, r, 0)),
            pl.BlockSpec((RB, D), lambda r: (r, 0)),
            pl.BlockSpec((RB, 1), lambda r: (r, 0)),
            pl.BlockSpec((D, D), lambda r: (0, 0)),
            pl.BlockSpec((1, D), lambda r: (0, 0)),
            pl.BlockSpec((D, OUT), lambda r: (0, 0)),
            pl.BlockSpec((1, OUT), lambda r: (0, 0)),
        ],
        out_specs=pl.BlockSpec((RB, OUT), lambda r: (r, 0)),
        out_shape=jax.ShapeDtypeStruct((N, OUT), jnp.float32),
    )(tp, g, dinv, w3, b3, w4, b4)


def kernel(x, edge_index, W1, b1, W2, b2, W3, b3, W4, b4):
    src3 = edge_index[0].reshape(NT, NCH, GW, PKW)
    dst3 = edge_index[1].reshape(NT, NCH, GW, PKW)
    dst3d = edge_index[1].reshape(NT, NWIN, KW)
    zflat = jnp.zeros((DSL,), jnp.float32)
    zrows = jnp.zeros((RPT, D), jnp.float32)

    degp = _deg_call(dst3d, zflat)
    g, dinv, dinv2 = _prep(degp.reshape(NC, NPAD, 1), x)

    for r, (Wr, br) in enumerate(((W1, b1), (W2, b2), (W3, b3))):
        for _ in range(2):
            tp = _prop_call(g, src3, dst3, zrows)
            g = _mid(tp, g, dinv2)
        tp = _prop_call(g, src3, dst3, zrows)
        if r < 2:
            g = _round(tp, g, dinv, Wr, br.reshape(1, D))
        else:
            out = _final(tp, g, dinv, W3, b3.reshape(1, D), W4, b4.reshape(1, OUT))
    return out


# speedup vs baseline: 1.2581x; 1.2581x over previous
import jax
import jax.numpy as jnp
from jax import lax
from jax.experimental import pallas as pl
from jax.experimental.pallas import tpu as pltpu
from jax.experimental.pallas import tpu_sc as plsc

N = 10000
E = 320000
D = 128
OUT = 1024

NC = 2
NS = 16
NT = NC * NS
EPT = E // NT
KW = 125
NWIN = EPT // KW
PKW = 80
GW = 25
NCH = EPT // (GW * PKW)
NB = 3
LA = 2
NRT = 10240
RPT = NRT // NS
NPAD = 10240
DSL = NPAD // NS

_mesh = plsc.VectorSubcoreMesh(
    core_axis_name="c", subcore_axis_name="s", num_cores=NC, num_subcores=NS
)


def _deg_body(dsts_hbm, zflat_hbm, out_hbm, dstv, onesv, dsem, deg_sh):
    c = lax.axis_index("c")
    s = lax.axis_index("s")
    chunk = c * NS + s
    pltpu.sync_copy(dsts_hbm.at[chunk], dstv)
    for i in range(128 // 16):
        onesv[pl.ds(i * 16, 16)] = jnp.ones((16,), jnp.float32)
    pltpu.sync_copy(zflat_hbm, deg_sh.at[pl.ds(s * DSL, DSL)])
    plsc.subcore_barrier()

    ones_kw = onesv.at[pl.ds(0, KW)]

    def fire(w, tok):
        pltpu.async_copy(ones_kw, deg_sh.at[dstv.at[w]], dsem, add=True)
        return tok

    def drain(w, tok):
        pltpu.make_async_copy(ones_kw, deg_sh.at[dstv.at[w]], dsem).wait()
        return tok

    lax.fori_loop(0, NWIN, fire, 0)
    lax.fori_loop(0, NWIN, drain, 0)
    plsc.subcore_barrier()
    pltpu.sync_copy(
        deg_sh.at[pl.ds(s * DSL, DSL)], out_hbm.at[c].at[pl.ds(s * DSL, DSL)]
    )


_deg_call = pl.kernel(
    _deg_body,
    out_type=jax.ShapeDtypeStruct((NC, NPAD), jnp.float32),
    mesh=_mesh,
    scratch_types=[
        pltpu.VMEM((NWIN, KW), jnp.int32),
        pltpu.VMEM((128,), jnp.float32),
        pltpu.SemaphoreType.DMA,
        pltpu.VMEM_SHARED((NPAD,), jnp.float32),
    ],
)


def _prop_body(g_hbm, srcs_hbm, dsts_hbm, zrows_hbm, out_hbm, sbuf, dbuf, buf, gsem, ssem, acc):
    c = lax.axis_index("c")
    s = lax.axis_index("s")
    tchunk = c * NS + s
    base = s * RPT
    pltpu.sync_copy(zrows_hbm, acc.at[pl.ds(base, RPT)])
    plsc.subcore_barrier()

    def gissue(wl, b):
        pltpu.async_copy(g_hbm.at[sbuf.at[wl]], buf.at[b], gsem.at[b])

    def gwait(wl, b):
        pltpu.make_async_copy(g_hbm.at[sbuf.at[wl]], buf.at[b], gsem.at[b]).wait()

    def sissue(wl, b):
        pltpu.async_copy(buf.at[b], acc.at[dbuf.at[wl]], ssem.at[b], add=True)

    def swait(wl, b):
        pltpu.make_async_copy(buf.at[b], acc.at[dbuf.at[wl]], ssem.at[b]).wait()

    def chunk_body(j, tok):
        pltpu.sync_copy(srcs_hbm.at[tchunk].at[j], sbuf)
        pltpu.sync_copy(dsts_hbm.at[tchunk].at[j], dbuf)
        for p in range(LA):
            gissue(p, p % NB)
        for wl in range(GW):
            b = wl % NB
            gwait(wl, b)
            sissue(wl, b)
            v = wl + LA
            if v < GW:
                bv = v % NB
                if v >= NB:
                    swait(v - NB, bv)
                gissue(v, bv)
        for wl in range(GW - NB, GW):
            if wl >= 0:
                swait(wl, wl % NB)
        return tok

    lax.fori_loop(0, NCH, chunk_body, 0)
    plsc.subcore_barrier()
    pltpu.sync_copy(acc.at[pl.ds(base, RPT)], out_hbm.at[c].at[pl.ds(base, RPT)])


_prop_call = pl.kernel(
    _prop_body,
    out_type=jax.ShapeDtypeStruct((NC, NRT, D), jnp.float32),
    mesh=_mesh,
    scratch_types=[
        pltpu.VMEM((GW, PKW), jnp.int32),
        pltpu.VMEM((GW, PKW), jnp.int32),
        pltpu.VMEM((NB, PKW, D), jnp.float32),
        pltpu.SemaphoreType.DMA((NB,)),
        pltpu.SemaphoreType.DMA((NB,)),
        pltpu.VMEM_SHARED((NRT, D), jnp.float32),
    ],
)


def _prep_kernel(degp_ref, x_ref, g_ref, dinv_ref, dinv2_ref):
    degb = degp_ref[0] + degp_ref[1] + 1.0
    dinv = lax.rsqrt(degb)
    dinv_ref[...] = dinv
    dinv2_ref[...] = 1.0 / degb
    g_ref[...] = dinv * x_ref[...]


def _prep(degp, x):
    RB = 400
    grid = N // RB
    return pl.pallas_call(
        _prep_kernel,
        grid=(grid,),
        in_specs=[
            pl.BlockSpec((NC, RB, 1), lambda r: (0, r, 0)),
            pl.BlockSpec((RB, D), lambda r: (r, 0)),
        ],
        out_specs=[
            pl.BlockSpec((RB, D), lambda r: (r, 0)),
            pl.BlockSpec((RB, 1), lambda r: (r, 0)),
            pl.BlockSpec((RB, 1), lambda r: (r, 0)),
        ],
        out_shape=[
            jax.ShapeDtypeStruct((N, D), jnp.float32),
            jax.ShapeDtypeStruct((N, 1), jnp.float32),
            jax.ShapeDtypeStruct((N, 1), jnp.float32),
        ],
    )(degp, x)


def _mid_kernel(tp_ref, g_ref, dinv2_ref, o_ref):
    o_ref[...] = dinv2_ref[...] * (tp_ref[0] + tp_ref[1] + g_ref[...])


def _mid(tp, g, dinv2):
    RB = 400
    grid = N // RB
    return pl.pallas_call(
        _mid_kernel,
        grid=(grid,),
        in_specs=[
            pl.BlockSpec((NC, RB, D), lambda r: (0, r, 0)),
            pl.BlockSpec((RB, D), lambda r: (r, 0)),
            pl.BlockSpec((RB, 1), lambda r: (r, 0)),
        ],
        out_specs=pl.BlockSpec((RB, D), lambda r: (r, 0)),
        out_shape=jax.ShapeDtypeStruct((N, D), jnp.float32),
    )(tp, g, dinv2)


def _round_kernel(tp_ref, g_ref, dinv_ref, w_ref, b_ref, o_ref):
    dinv = dinv_ref[...]
    h = dinv * (tp_ref[0] + tp_ref[1] + g_ref[...])
    hw = jnp.dot(h, w_ref[...], preferred_element_type=jnp.float32) + b_ref[...]
    o_ref[...] = dinv * hw


def _round(tp, g, dinv, w, b):
    RB = 1000
    grid = N // RB
    return pl.pallas_call(
        _round_kernel,
        grid=(grid,),
        in_specs=[
            pl.BlockSpec((NC, RB, D), lambda r: (0, r, 0)),
            pl.BlockSpec((RB, D), lambda r: (r, 0)),
            pl.BlockSpec((RB, 1), lambda r: (r, 0)),
            pl.BlockSpec((D, D), lambda r: (0, 0)),
            pl.BlockSpec((1, D), lambda r: (0, 0)),
        ],
        out_specs=pl.BlockSpec((RB, D), lambda r: (r, 0)),
        out_shape=jax.ShapeDtypeStruct((N, D), jnp.float32),
    )(tp, g, dinv, w, b)


def _final_kernel(tp_ref, g_ref, dinv_ref, w3_ref, b3_ref, w4_ref, b4_ref, o_ref):
    h = dinv_ref[...] * (tp_ref[0] + tp_ref[1] + g_ref[...])
    h3 = jnp.dot(h, w3_ref[...], preferred_element_type=jnp.float32) + b3_ref[...]
    o_ref[...] = (
        jnp.dot(h3, w4_ref[...], preferred_element_type=jnp.float32) + b4_ref[...]
    )


def _final(tp, g, dinv, w3, b3, w4, b4):
    RB = 400
    grid = N // RB
    return pl.pallas_call(
        _final_kernel,
        grid=(grid,),
        in_specs=[
            pl.BlockSpec((NC, RB, D), lambda r: (0, r, 0)),
            pl.BlockSpec((RB, D), lambda r: (r, 0)),
            pl.BlockSpec((RB, 1), lambda r: (r, 0)),
            pl.BlockSpec((D, D), lambda r: (0, 0)),
            pl.BlockSpec((1, D), lambda r: (0, 0)),
            pl.BlockSpec((D, OUT), lambda r: (0, 0)),
            pl.BlockSpec((1, OUT), lambda r: (0, 0)),
        ],
        out_specs=pl.BlockSpec((RB, OUT), lambda r: (r, 0)),
        out_shape=jax.ShapeDtypeStruct((N, OUT), jnp.float32),
    )(tp, g, dinv, w3, b3, w4, b4)


def kernel(x, edge_index, W1, b1, W2, b2, W3, b3, W4, b4):
    src3 = edge_index[0].reshape(NT, NCH, GW, PKW)
    dst3 = edge_index[1].reshape(NT, NCH, GW, PKW)
    dst3d = edge_index[1].reshape(NT, NWIN, KW)
    zflat = jnp.zeros((DSL,), jnp.float32)
    zrows = jnp.zeros((RPT, D), jnp.float32)

    degp = _deg_call(dst3d, zflat)
    g, dinv, dinv2 = _prep(degp.reshape(NC, NPAD, 1), x)

    for r, (Wr, br) in enumerate(((W1, b1), (W2, b2), (W3, b3))):
        for _ in range(2):
            tp = _prop_call(g, src3, dst3, zrows)
            g = _mid(tp, g, dinv2)
        tp = _prop_call(g, src3, dst3, zrows)
        if r < 2:
            g = _round(tp, g, dinv, Wr, br.reshape(1, D))
        else:
            out = _final(tp, g, dinv, W3, b3.reshape(1, D), W4, b4.reshape(1, OUT))
    return out
